# Initial kernel scaffold; baseline (speedup 1.0000x reference)
import jax
import jax.numpy as jnp
from jax.experimental import pallas as pl


def kernel(zL, params):
    raise NotImplementedError("write your pallas kernel here")


# speedup vs baseline: 7.3724x; 7.3724x over previous
import functools

import jax
import jax.numpy as jnp
import numpy as np
from jax.experimental import pallas as pl
from jax.experimental.pallas import tpu as pltpu

_B = 1
_DEPTH = 16
_L = 512
_H = 768
_NH = 12
_HD = _H // _NH
_F = 4096
_K = 64
_SCALE = 1.0 / np.sqrt(_HD)
_EPS = 1e-5

_ROWS_C = 256


_BF = jnp.bfloat16
_HI = jax.lax.Precision.HIGHEST
_RECIP_H = np.float32(1.0 / _H)


def _mm(a, b):
    return jax.lax.dot_general(a.astype(_BF), b.astype(_BF),
                               (((1,), (0,)), ((), ())),
                               preferred_element_type=jnp.float32)


def _dot(a, b, dims):
    return jax.lax.dot_general(a.astype(_BF), b.astype(_BF), dims,
                               preferred_element_type=jnp.float32)


def _lattn_body(x_ref, xn_ref, wq_ref, wk_ref, wv_ref, wo_ref, out_ref):
    x = x_ref[0]
    xn = xn_ref[0]
    q = _mm(xn, wq_ref[...]).astype(_BF)
    k = _mm(xn, wk_ref[...]).astype(_BF)
    v = _mm(xn, wv_ref[...])
    outs = []
    for h in range(_NH):
        sl = slice(h * _HD, (h + 1) * _HD)
        qh, kh, vh = q[:, sl], k[:, sl], v[:, sl]
        s = _dot(qh, kh, (((1,), (1,)), ((), ()))) * _SCALE
        m = jnp.max(s, axis=-1, keepdims=True)
        p = jnp.exp(s - m)
        outs.append(_mm(p, vh) / jnp.sum(p, axis=-1, keepdims=True))
    o = jnp.concatenate(outs, axis=1)
    out_ref[0] = x + _mm(o, wo_ref[...])


def _dattn_body(y_ref, yn_ref, wq_ref, wk_ref, wv_ref, wo_ref, out_ref):
    y = y_ref[...]
    bl = y.shape[0]
    yn = yn_ref[...].astype(_BF)
    flat = yn.astype(jnp.float32).reshape(bl * _DEPTH, _H)
    q = _mm(flat, wq_ref[...]).astype(_BF).reshape(bl, _DEPTH, _H)
    k = _mm(flat, wk_ref[...]).astype(_BF).reshape(bl, _DEPTH, _H)
    v = _mm(flat, wv_ref[...]).astype(_BF).reshape(bl, _DEPTH, _H)
    rows = jax.lax.broadcasted_iota(jnp.int32, (_DEPTH, _DEPTH), 0)
    cols = jax.lax.broadcasted_iota(jnp.int32, (_DEPTH, _DEPTH), 1)
    causal = rows >= cols
    outs = []
    for h in range(_NH):
        sl = slice(h * _HD, (h + 1) * _HD)
        qh, kh, vh = q[:, :, sl], k[:, :, sl], v[:, :, sl]
        s = _dot(qh, kh, (((2,), (2,)), ((0,), (0,)))) * _SCALE
        s = jnp.where(causal[None, :, :], s, -jnp.inf)
        m = jnp.max(s, axis=-1, keepdims=True)
        p = jnp.exp(s - m)
        pn = p / jnp.sum(p, axis=-1, keepdims=True)
        outs.append(_dot(pn, vh, (((2,), (1,)), ((0,), (0,)))).astype(_BF))
    o = jnp.concatenate(outs, axis=2).astype(jnp.float32).reshape(
        bl * _DEPTH, _H)
    out_ref[...] = y + _mm(o, wo_ref[...]).reshape(bl, _DEPTH, _H)


def _enc_body(z_ref, p_ref, de_ref, bp_ref, be_ref, out_ref):
    xs = z_ref[...] - p_ref[...] - bp_ref[...]
    logits = _dot(xs, de_ref[...], (((1,), (1,)), ((), ())))
    z = jnp.maximum(logits + be_ref[...], 0.0)
    zi = jax.lax.bitcast_convert_type(z, jnp.int32)
    r = z.shape[0]
    one = jnp.int32(1)

    def vstep(i, t):
        cand = t | jax.lax.shift_left(one, 30 - i)
        cnt = jnp.sum((zi >= cand).astype(jnp.int32), axis=1, keepdims=True)
        return jnp.where(cnt >= _K, cand, t)

    t0 = jnp.zeros((r, 1), jnp.int32)
    tv = jax.lax.fori_loop(0, 31, vstep, t0)

    gt = zi > tv
    cnt_gt = jnp.sum(gt.astype(jnp.int32), axis=1, keepdims=True)
    needed = _K - cnt_gt
    eq = zi == tv
    idx = jax.lax.broadcasted_iota(jnp.int32, (r, _F), 1)

    def istep(i, c):
        cand = c | jax.lax.shift_left(one, 11 - i)
        cnt = jnp.sum((eq & (idx < cand)).astype(jnp.int32), axis=1,
                      keepdims=True)
        return jnp.where(cnt < needed, cand, c)

    ci = jax.lax.fori_loop(0, 12, istep, t0)
    keep = gt | (eq & (idx <= ci))
    out_ref[...] = jnp.where(keep, z, 0.0)


@functools.partial(jax.jit, static_argnames=())
def kernel(zL, params):
    p = params
    B, D, L, H = zL.shape
    f32 = jnp.float32

    qtok = jnp.broadcast_to(p['query_token'][None, None, None, :], (B, 1, L, H))
    x = jnp.concatenate([qtok, zL[:, :-1]], axis=1).reshape(B * D, L, H)

    def _ln(t, w, b):
        mu = jnp.mean(t, axis=-1, keepdims=True)
        var = jnp.mean((t - mu) ** 2, axis=-1, keepdims=True)
        return (t - mu) / jnp.sqrt(var + _EPS) * w + b

    xn = _ln(x, p['norm_l_w'], p['norm_l_b'])

    wspec = pl.BlockSpec((H, H), lambda i: (0, 0))
    xspec = pl.BlockSpec((1, L, H), lambda i: (i, 0, 0))

    x1 = pl.pallas_call(
        _lattn_body,
        grid=(B * D,),
        in_specs=[xspec, xspec, wspec, wspec, wspec, wspec],
        out_specs=xspec,
        out_shape=jax.ShapeDtypeStruct((B * D, L, H), f32),
    )(x, xn, p['l_q'].T, p['l_k'].T, p['l_v'].T, p['l_o'].T)

    y = x1.reshape(B, D, L, H).transpose(0, 2, 1, 3).reshape(B * L, D, H)
    yn = _ln(y, p['norm_d_w'], p['norm_d_b'])

    BL = 64
    yspec = pl.BlockSpec((BL, D, H), lambda i: (i, 0, 0))
    y1 = pl.pallas_call(
        _dattn_body,
        grid=(B * L // BL,),
        in_specs=[yspec, yspec, wspec, wspec, wspec, wspec],
        out_specs=yspec,
        out_shape=jax.ShapeDtypeStruct((B * L, D, H), f32),
    )(y, yn, p['d_q'].T, p['d_k'].T, p['d_v'].T, p['d_o'].T)

    prior = y1.reshape(B, L, D, H).transpose(0, 2, 1, 3).reshape(B * D * L, H)
    zrows = zL.reshape(B * D * L, H)
    N = B * D * L
    R = _ROWS_C

    out = pl.pallas_call(
        _enc_body,
        grid=(N // R,),
        in_specs=[
            pl.BlockSpec((R, H), lambda i: (i, 0)),
            pl.BlockSpec((R, H), lambda i: (i, 0)),
            pl.BlockSpec((_F, H), lambda i: (0, 0)),
            pl.BlockSpec((1, H), lambda i: (0, 0)),
            pl.BlockSpec((1, _F), lambda i: (0, 0)),
        ],
        out_specs=pl.BlockSpec((R, _F), lambda i: (i, 0)),
        out_shape=jax.ShapeDtypeStruct((N, _F), f32),
    )(zrows, prior, p['dict_enc'], p['bias_pre'].reshape(1, H),
      p['bias_enc'].reshape(1, _F))

    return out.reshape(B, D, L, _F)
